# baseline (speedup 1.0000x reference)
import dataclasses
import functools

import jax
import jax.numpy as jnp
from jax import lax
from jax.experimental import pallas as pl
from jax.experimental.pallas import tpu as pltpu
from jax.experimental.pallas import tpu_sc as plsc

N_NODES = 4096
X_DIM = 64
PAIR = 2 * X_DIM
N_IN = 262144

NUM_CORES = 2
NUM_SUBCORES = 16
NW = NUM_CORES * NUM_SUBCORES
LANES = 16

CH = 128
QPW = N_IN // NW
NCHUNK = QPW // CH


def _compiler_params():
    cp = pltpu.CompilerParams()
    if "needs_layout_passes" in pltpu.CompilerParams.__dataclass_fields__:
        cp = dataclasses.replace(cp, needs_layout_passes=False)
    return cp


NBUF = 2


def _sc_interp(x_in, y_pair):
    mesh = plsc.VectorSubcoreMesh(core_axis_name="c", subcore_axis_name="s")

    FT = X_DIM // 8
    QT = N_IN // CH

    @functools.partial(
        pl.kernel,
        mesh=mesh,
        compiler_params=_compiler_params(),
        out_type=jax.ShapeDtypeStruct((FT, QT, 8, CH), jnp.float32),
        scratch_types=[
            pltpu.VMEM((QPW,), jnp.float32),
            pltpu.VMEM((NCHUNK, CH), jnp.int32),
            pltpu.VMEM((NCHUNK, CH), jnp.float32),
            pltpu.VMEM((NBUF, CH, PAIR), jnp.float32),
            pltpu.VMEM((NBUF, FT, 8, CH), jnp.float32),
        ]
        + [pltpu.SemaphoreType.DMA] * (2 * NBUF),
    )
    def k(x_hbm, pair_hbm, out_hbm, x_v, idx_v, t_v, rows_v, o_v, *sems):
        gsem = sems[:NBUF]
        wsem = sems[NBUF:]
        wid = lax.axis_index("s") * NUM_CORES + lax.axis_index("c")
        tile0 = wid * QPW

        pltpu.sync_copy(x_hbm.at[pl.ds(tile0, QPW)], x_v)

        @pl.loop(0, NCHUNK)
        def _pre(c):
            @pl.loop(0, CH, step=LANES)
            def _idx(g):
                xv = x_v[pl.ds(c * CH + g, LANES)]
                i = jnp.minimum(
                    lax.convert_element_type(xv, jnp.int32), N_NODES - 2
                )
                idx_v[c, pl.ds(g, LANES)] = i
                t_v[c, pl.ds(g, LANES)] = xv - lax.convert_element_type(
                    i, jnp.float32
                )

        def fire(cc, b):
            pltpu.async_copy(pair_hbm.at[idx_v.at[cc]], rows_v.at[b], gsem[b])

        def lerp(cc, b):
            rows2 = rows_v.at[b]
            ts = []
            oms = []
            qis = []
            for kk in range(CH // LANES):
                tv = t_v[cc, pl.ds(kk * LANES, LANES)]
                ts.append(tv)
                oms.append(1.0 - tv)
                qis.append(lax.iota(jnp.int32, LANES) + (kk * LANES))
            for f in range(X_DIM):
                ca = jnp.full((LANES,), f, jnp.int32)
                cb = jnp.full((LANES,), f + X_DIM, jnp.int32)
                i, s = f // 8, f % 8
                for kk in range(CH // LANES):
                    a = plsc.load_gather(rows2, [qis[kk], ca])
                    bb = plsc.load_gather(rows2, [qis[kk], cb])
                    o_v[b, i, s, pl.ds(kk * LANES, LANES)] = (
                        a * oms[kk] + bb * ts[kk]
                    )

        for b in range(NBUF):
            fire(b, b)

        @pl.loop(0, NCHUNK, step=NBUF)
        def _chunks(c):
            for b in range(NBUF):
                cc = c + b
                pltpu.make_async_copy(
                    pair_hbm.at[idx_v.at[cc]], rows_v.at[b], gsem[b]
                ).wait()

                @pl.when(cc >= NBUF)
                def _():
                    for i in range(FT):
                        pltpu.make_async_copy(
                            o_v.at[b, i], out_hbm.at[i, 0], wsem[b]
                        ).wait()

                lerp(cc, b)
                jc = wid * NCHUNK + cc
                for i in range(FT):
                    pltpu.async_copy(
                        o_v.at[b, i], out_hbm.at[i, jc], wsem[b]
                    )

                @pl.when(cc + NBUF < NCHUNK)
                def _():
                    fire(cc + NBUF, b)

        for b in range(NBUF):
            for i in range(FT):
                pltpu.make_async_copy(
                    o_v.at[b, i], out_hbm.at[i, 0], wsem[b]
                ).wait()

    return k(x_in, y_pair)


@jax.jit
def kernel(x_in, x_node, y_node):
    del x_node
    x_in = x_in.ravel()
    y_pair = jnp.concatenate([y_node[:-1], y_node[1:]], axis=1)
    tiles = _sc_interp(x_in, y_pair)
    return tiles.transpose(1, 3, 0, 2).reshape(N_IN, X_DIM)


# speedup vs baseline: 1.2307x; 1.2307x over previous
import dataclasses
import functools

import jax
import jax.numpy as jnp
from jax import lax
from jax.experimental import pallas as pl
from jax.experimental.pallas import tpu as pltpu
from jax.experimental.pallas import tpu_sc as plsc

N_NODES = 4096
X_DIM = 64
PAIR = 2 * X_DIM
N_IN = 262144

NUM_CORES = 2
NUM_SUBCORES = 16
NW = NUM_CORES * NUM_SUBCORES
LANES = 16

CH = 128
QPW = N_IN // NW
NCHUNK = QPW // CH


def _compiler_params():
    cp = pltpu.CompilerParams()
    if "needs_layout_passes" in pltpu.CompilerParams.__dataclass_fields__:
        cp = dataclasses.replace(cp, needs_layout_passes=False)
    return cp


NBUF = 2


def _sc_interp(x_in, y_pair):
    mesh = plsc.VectorSubcoreMesh(core_axis_name="c", subcore_axis_name="s")

    FT = X_DIM // 8
    QT = N_IN // CH

    @functools.partial(
        pl.kernel,
        mesh=mesh,
        compiler_params=_compiler_params(),
        out_type=jax.ShapeDtypeStruct((FT, QT, 8, CH), jnp.float32),
        scratch_types=[
            pltpu.VMEM((QPW,), jnp.float32),
            pltpu.VMEM((NCHUNK, CH), jnp.int32),
            pltpu.VMEM((NCHUNK, CH), jnp.float32),
            pltpu.VMEM((NBUF, CH, PAIR), jnp.float32),
            pltpu.VMEM((NBUF, CH, X_DIM + 1), jnp.float32),
            pltpu.VMEM((NBUF, FT, 8, CH), jnp.float32),
        ]
        + [pltpu.SemaphoreType.DMA] * (2 * NBUF),
    )
    def k(x_hbm, pair_hbm, out_hbm, x_v, idx_v, t_v, rows_v, pad_v, o_v,
          *sems):
        gsem = sems[:NBUF]
        wsem = sems[NBUF:]
        wid = lax.axis_index("s") * NUM_CORES + lax.axis_index("c")
        tile0 = wid * QPW

        pltpu.sync_copy(x_hbm.at[pl.ds(tile0, QPW)], x_v)

        @pl.loop(0, NCHUNK)
        def _pre(c):
            @pl.loop(0, CH, step=LANES)
            def _idx(g):
                xv = x_v[pl.ds(c * CH + g, LANES)]
                i = jnp.minimum(
                    lax.convert_element_type(xv, jnp.int32), N_NODES - 2
                )
                idx_v[c, pl.ds(g, LANES)] = i
                t_v[c, pl.ds(g, LANES)] = xv - lax.convert_element_type(
                    i, jnp.float32
                )

        def fire(cc, b):
            pltpu.async_copy(pair_hbm.at[idx_v.at[cc]], rows_v.at[b], gsem[b])

        def lerp(cc, b):
            pad2 = pad_v.at[b]
            col_iota = [
                lax.iota(jnp.int32, LANES) + cg * LANES
                for cg in range(X_DIM // LANES)
            ]

            @pl.loop(0, CH // LANES)
            def _lerp(k):
                g = k * LANES
                t16 = t_v[cc, pl.ds(g, LANES)]
                for q in range(LANES):
                    row = g + q
                    rsplat = jnp.full((LANES,), row, jnp.int32)
                    tq = lax.gather(
                        t16,
                        jnp.full((LANES, 1), q, jnp.int32),
                        lax.GatherDimensionNumbers(
                            offset_dims=(),
                            collapsed_slice_dims=(0,),
                            start_index_map=(0,),
                        ),
                        (1,),
                        mode=lax.GatherScatterMode.PROMISE_IN_BOUNDS,
                    )
                    om = 1.0 - tq
                    for cg in range(X_DIM // LANES):
                        a = rows_v[b, row, pl.ds(cg * LANES, LANES)]
                        bb = rows_v[b, row, pl.ds(X_DIM + cg * LANES, LANES)]
                        plsc.store_scatter(
                            pad2, [rsplat, col_iota[cg]], a * om + bb * tq
                        )

            qis = [
                lax.iota(jnp.int32, LANES) + kk * LANES
                for kk in range(CH // LANES)
            ]
            for f in range(X_DIM):
                cf = jnp.full((LANES,), f, jnp.int32)
                i, s = f // 8, f % 8
                for kk in range(CH // LANES):
                    o_v[b, i, s, pl.ds(kk * LANES, LANES)] = plsc.load_gather(
                        pad2, [qis[kk], cf]
                    )

        for b in range(NBUF):
            fire(b, b)

        @pl.loop(0, NCHUNK, step=NBUF)
        def _chunks(c):
            for b in range(NBUF):
                cc = c + b
                pltpu.make_async_copy(
                    pair_hbm.at[idx_v.at[cc]], rows_v.at[b], gsem[b]
                ).wait()

                @pl.when(cc >= NBUF)
                def _():
                    for i in range(FT):
                        pltpu.make_async_copy(
                            o_v.at[b, i], out_hbm.at[i, 0], wsem[b]
                        ).wait()

                lerp(cc, b)
                jc = wid * NCHUNK + cc
                for i in range(FT):
                    pltpu.async_copy(
                        o_v.at[b, i], out_hbm.at[i, jc], wsem[b]
                    )

                @pl.when(cc + NBUF < NCHUNK)
                def _():
                    fire(cc + NBUF, b)

        for b in range(NBUF):
            for i in range(FT):
                pltpu.make_async_copy(
                    o_v.at[b, i], out_hbm.at[i, 0], wsem[b]
                ).wait()

    return k(x_in, y_pair)


@jax.jit
def kernel(x_in, x_node, y_node):
    del x_node
    x_in = x_in.ravel()
    y_pair = jnp.concatenate([y_node[:-1], y_node[1:]], axis=1)
    tiles = _sc_interp(x_in, y_pair)
    return tiles.transpose(1, 3, 0, 2).reshape(N_IN, X_DIM)


# speedup vs baseline: 2.6682x; 2.1680x over previous
import dataclasses
import functools

import jax
import jax.numpy as jnp
from jax import lax
from jax.experimental import pallas as pl
from jax.experimental.pallas import tpu as pltpu
from jax.experimental.pallas import tpu_sc as plsc

N_NODES = 4096
X_DIM = 64
PAIR = 2 * X_DIM
N_IN = 262144

NUM_CORES = 2
NUM_SUBCORES = 16
NW = NUM_CORES * NUM_SUBCORES
LANES = 16

CH = 128
NBUF = 2
NSLICE = 4

QPS = N_IN // NSLICE
QPW = QPS // NW
NCHUNK = QPW // CH


def _compiler_params():
    cp = pltpu.CompilerParams()
    if "needs_layout_passes" in pltpu.CompilerParams.__dataclass_fields__:
        cp = dataclasses.replace(cp, needs_layout_passes=False)
    return cp


def _sc_interp(x_in, y_pair):
    mesh = plsc.VectorSubcoreMesh(core_axis_name="c", subcore_axis_name="s")

    @functools.partial(
        pl.kernel,
        mesh=mesh,
        compiler_params=_compiler_params(),
        out_type=jax.ShapeDtypeStruct((QPS, X_DIM), jnp.float32),
        scratch_types=[
            pltpu.VMEM((QPW,), jnp.float32),
            pltpu.VMEM((NCHUNK, CH), jnp.int32),
            pltpu.VMEM((NCHUNK, CH), jnp.float32),
            pltpu.VMEM((NBUF, CH, PAIR), jnp.float32),
            pltpu.VMEM((NBUF, CH, X_DIM), jnp.float32),
        ]
        + [pltpu.SemaphoreType.DMA] * (2 * NBUF),
    )
    def k(x_hbm, pair_hbm, out_hbm, x_v, idx_v, t_v, rows_v, o_v, *sems):
        gsem = sems[:NBUF]
        wsem = sems[NBUF:]
        wid = lax.axis_index("s") * NUM_CORES + lax.axis_index("c")
        tile0 = wid * QPW

        pltpu.sync_copy(x_hbm.at[pl.ds(tile0, QPW)], x_v)

        @pl.loop(0, NCHUNK)
        def _pre(c):
            @pl.loop(0, CH, step=LANES)
            def _idx(g):
                xv = x_v[pl.ds(c * CH + g, LANES)]
                i = jnp.minimum(
                    lax.convert_element_type(xv, jnp.int32), N_NODES - 2
                )
                idx_v[c, pl.ds(g, LANES)] = i
                t_v[c, pl.ds(g, LANES)] = xv - lax.convert_element_type(
                    i, jnp.float32
                )

        def fire(cc, b):
            pltpu.async_copy(pair_hbm.at[idx_v.at[cc]], rows_v.at[b], gsem[b])

        def lerp(cc, b):
            @pl.loop(0, CH // LANES)
            def _lerp(k):
                g = k * LANES
                t16 = t_v[cc, pl.ds(g, LANES)]
                for q in range(LANES):
                    row = g + q
                    tq = lax.gather(
                        t16,
                        jnp.full((LANES, 1), q, jnp.int32),
                        lax.GatherDimensionNumbers(
                            offset_dims=(),
                            collapsed_slice_dims=(0,),
                            start_index_map=(0,),
                        ),
                        (1,),
                        mode=lax.GatherScatterMode.PROMISE_IN_BOUNDS,
                    )
                    om = 1.0 - tq
                    for cg in range(X_DIM // LANES):
                        a = rows_v[b, row, pl.ds(cg * LANES, LANES)]
                        bb = rows_v[b, row, pl.ds(X_DIM + cg * LANES, LANES)]
                        o_v[b, row, pl.ds(cg * LANES, LANES)] = (
                            a * om + bb * tq
                        )

        for b in range(min(NBUF, NCHUNK)):
            fire(b, b)

        @pl.loop(0, NCHUNK, step=NBUF)
        def _chunks(c):
            for b in range(NBUF):
                cc = c + b
                pltpu.make_async_copy(
                    pair_hbm.at[idx_v.at[cc]], rows_v.at[b], gsem[b]
                ).wait()

                @pl.when(cc >= NBUF)
                def _():
                    pltpu.make_async_copy(
                        o_v.at[b], out_hbm.at[pl.ds(tile0, CH)], wsem[b]
                    ).wait()

                lerp(cc, b)
                pltpu.async_copy(
                    o_v.at[b], out_hbm.at[pl.ds(tile0 + cc * CH, CH)], wsem[b]
                )

                @pl.when(cc + NBUF < NCHUNK)
                def _():
                    fire(cc + NBUF, b)

        for b in range(min(NBUF, NCHUNK)):
            pltpu.make_async_copy(
                o_v.at[b], out_hbm.at[pl.ds(tile0, CH)], wsem[b]
            ).wait()

    return k(x_in, y_pair)


@jax.jit
def kernel(x_in, x_node, y_node):
    del x_node
    x_in = x_in.ravel()
    y_pair = jnp.concatenate([y_node[:-1], y_node[1:]], axis=1)
    parts = [
        _sc_interp(lax.slice(x_in, (s * QPS,), ((s + 1) * QPS,)), y_pair)
        for s in range(NSLICE)
    ]
    return jnp.concatenate(parts, axis=0)
